# baseline (speedup 1.0000x reference)
import functools

import numpy as np
import jax
import jax.numpy as jnp
from jax import lax
from jax.experimental import pallas as pl
from jax.experimental.pallas import tpu as pltpu
from jax.experimental.pallas import tpu_sc as plsc

N_NODES = 10000
C = 128
N_EDGES = 320000
NPAD = 10240
NC, NS = 2, 16
NW = NC * NS
CHUNK = 128
CH = 80
SLAB = 16
GROUP = 2
NSLAB = CH // SLAB
EPT = CH * CHUNK
EPAD = NW * EPT
DUMMY = NPAD - 1
BN = 1280
STRIPE = NPAD // NS

def _mask_scale():
    keep = jax.random.bernoulli(jax.random.key(42), 1.0 - 0.2, (N_NODES, C))
    scale = jnp.float32(1.0) / jnp.float32(1.0 - 0.2)
    m = jnp.where(keep, scale, jnp.float32(0.0))
    return jnp.pad(m, ((0, NPAD - N_NODES), (0, 0)))


def _sc_mesh():
    return plsc.VectorSubcoreMesh(
        core_axis_name="c", subcore_axis_name="s", num_cores=NC, num_subcores=NS
    )


def _deg_partials(col3):
    @functools.partial(
        pl.kernel,
        out_type=jax.ShapeDtypeStruct((NW, NPAD), jnp.float32),
        mesh=_sc_mesh(),
        scratch_types=[
            pltpu.VMEM((CH, CHUNK), jnp.int32),
            pltpu.VMEM((NPAD,), jnp.float32),
        ],
        compiler_params=pltpu.CompilerParams(needs_layout_passes=False),
    )
    def k(col_hbm, out_hbm, colbuf, hist):
        cid = lax.axis_index("c")
        sid = lax.axis_index("s")
        wid = cid * NS + sid
        z16 = jnp.zeros((16,), jnp.float32)

        def zero_body(i, carry):
            hist[pl.ds(i * 16, 16)] = z16
            return carry

        lax.fori_loop(0, NPAD // 16, zero_body, 0)
        pltpu.sync_copy(col_hbm.at[wid], colbuf)
        ones16 = jnp.ones((16,), jnp.float32)

        def edge_body(e, carry):
            j = e // (CHUNK // 16)
            q = e % (CHUNK // 16)
            idx = colbuf[j, pl.ds(q * 16, 16)]
            plsc.addupdate_scatter(hist, [idx], ones16)
            return carry

        lax.fori_loop(0, CH * (CHUNK // 16), edge_body, 0)
        pltpu.sync_copy(hist, out_hbm.at[wid])

    return k(col3)


def _dense(xp, mp, W, parts):
    def body(x_ref, m_ref, w_ref, p_ref, h2_ref, dis_ref):
        ones = jnp.ones((NW, 1), jnp.float32)
        deg = (
            lax.dot_general(
                p_ref[...], ones, (((0,), (0,)), ((), ())),
                preferred_element_type=jnp.float32,
            )
            + 1.0
        )
        dis = lax.rsqrt(deg)
        h = jnp.dot(
            x_ref[...] * m_ref[...], w_ref[...],
            preferred_element_type=jnp.float32,
        )
        h2_ref[...] = dis * h
        dis_ref[...] = dis

    return pl.pallas_call(
        body,
        grid=(NPAD // BN,),
        in_specs=[
            pl.BlockSpec((BN, C), lambda i: (i, 0)),
            pl.BlockSpec((BN, C), lambda i: (i, 0)),
            pl.BlockSpec((C, C), lambda i: (0, 0)),
            pl.BlockSpec((NW, BN), lambda i: (0, i)),
        ],
        out_specs=[
            pl.BlockSpec((BN, C), lambda i: (i, 0)),
            pl.BlockSpec((BN, 1), lambda i: (i, 0)),
        ],
        out_shape=[
            jax.ShapeDtypeStruct((NPAD, C), jnp.float32),
            jax.ShapeDtypeStruct((NPAD, 1), jnp.float32),
        ],
    )(xp, mp, W, parts)


def _edge_accumulate(h2, row3, col3):
    @functools.partial(
        pl.kernel,
        out_type=jax.ShapeDtypeStruct((NC, NPAD, C), jnp.float32),
        mesh=_sc_mesh(),
        scratch_types=[
            pltpu.VMEM((CH, CHUNK), jnp.int32),
            pltpu.VMEM((CH, CHUNK), jnp.int32),
            pltpu.VMEM((CHUNK, C), jnp.float32),
            pltpu.VMEM_SHARED((NPAD, C), jnp.float32),
            pltpu.SemaphoreType.DMA,
        ],
        compiler_params=pltpu.CompilerParams(needs_layout_passes=False),
    )
    def k(h2_hbm, row_hbm, col_hbm, out_hbm, rowbuf, colbuf, gbuf, acc, sem):
        cid = lax.axis_index("c")
        sid = lax.axis_index("s")
        wid = cid * NS + sid
        z16 = jnp.zeros((16,), jnp.float32)

        def zbuf_body(i, carry):
            r = i // (C // 16)
            q = i % (C // 16)
            gbuf[r, pl.ds(q * 16, 16)] = z16
            return carry

        lax.fori_loop(0, CHUNK * (C // 16), zbuf_body, 0)

        def zacc_body(i, carry):
            pltpu.sync_copy(gbuf, acc.at[pl.ds(sid * STRIPE + i * CHUNK, CHUNK)])
            return carry

        lax.fori_loop(0, STRIPE // CHUNK, zacc_body, 0)
        pltpu.sync_copy(row_hbm.at[wid], rowbuf)
        pltpu.sync_copy(col_hbm.at[wid], colbuf)
        plsc.subcore_barrier()

        def edge_body(j, carry):
            pltpu.async_copy(h2_hbm.at[rowbuf.at[j]], gbuf, sem).wait()
            pltpu.sync_copy(gbuf, acc.at[colbuf.at[j]], add=True)
            return carry

        lax.fori_loop(0, CH, edge_body, 0)
        plsc.subcore_barrier()
        pltpu.sync_copy(
            acc.at[pl.ds(sid * STRIPE, STRIPE)],
            out_hbm.at[cid, pl.ds(sid * STRIPE, STRIPE)],
        )

    return k(h2, row3, col3)


def _finalize(acc_parts, h2, dis, b2):
    def body(a_ref, h2_ref, dis_ref, b_ref, y_ref):
        s = a_ref[0] + a_ref[1] + h2_ref[...]
        out = jnp.maximum(dis_ref[...] * s + b_ref[...], 0.0)
        nrm = jnp.sqrt(jnp.sum(out * out, axis=1, keepdims=True))
        y_ref[...] = out / jnp.maximum(nrm, 1e-12)

    return pl.pallas_call(
        body,
        grid=(NPAD // BN,),
        in_specs=[
            pl.BlockSpec((NC, BN, C), lambda i: (0, i, 0)),
            pl.BlockSpec((BN, C), lambda i: (i, 0)),
            pl.BlockSpec((BN, 1), lambda i: (i, 0)),
            pl.BlockSpec((1, C), lambda i: (0, 0)),
        ],
        out_specs=pl.BlockSpec((BN, C), lambda i: (i, 0)),
        out_shape=jax.ShapeDtypeStruct((NPAD, C), jnp.float32),
    )(acc_parts, h2, dis, b2)


def kernel(x, edge_index, W, b):
    xp = jnp.pad(x.astype(jnp.float32), ((0, NPAD - N_NODES), (0, 0)))
    mp = _mask_scale()
    rowp = jnp.concatenate(
        [edge_index[0], jnp.zeros((EPAD - N_EDGES,), jnp.int32)]
    ).reshape(NW, CH, CHUNK)
    pad_cols = N_NODES + jnp.arange(EPAD - N_EDGES, dtype=jnp.int32) % (
        NPAD - N_NODES
    )
    colp = jnp.concatenate([edge_index[1], pad_cols]).reshape(NW, CH, CHUNK)

    parts = _deg_partials(colp)
    h2, dis = _dense(xp, mp, W.astype(jnp.float32), parts)
    acc_parts = _edge_accumulate(h2, rowp, colp)
    y = _finalize(acc_parts, h2, dis, b.reshape(1, C).astype(jnp.float32))
    return y[:N_NODES]


# speedup vs baseline: 2.1650x; 2.1650x over previous
import functools

import numpy as np
import jax
import jax.numpy as jnp
from jax import lax
from jax.experimental import pallas as pl
from jax.experimental.pallas import tpu as pltpu
from jax.experimental.pallas import tpu_sc as plsc

N_NODES = 10000
C = 128
N_EDGES = 320000
NPAD = 10240
NC, NS = 2, 16
NW = NC * NS
CHUNK = 128
CH = 80
SLAB = 16
GROUP = 2
NSLAB = CH // SLAB
EPT = CH * CHUNK
EPAD = NW * EPT
DUMMY = NPAD - 1
BN = 1280
STRIPE = NPAD // NS

def _mask_scale():
    keep = jax.random.bernoulli(jax.random.key(42), 1.0 - 0.2, (N_NODES, C))
    scale = jnp.float32(1.0) / jnp.float32(1.0 - 0.2)
    m = jnp.where(keep, scale, jnp.float32(0.0))
    return jnp.pad(m, ((0, NPAD - N_NODES), (0, 0)))


def _sc_mesh():
    return plsc.VectorSubcoreMesh(
        core_axis_name="c", subcore_axis_name="s", num_cores=NC, num_subcores=NS
    )


def _deg_partials(col3):
    @functools.partial(
        pl.kernel,
        out_type=jax.ShapeDtypeStruct((NW, NPAD), jnp.float32),
        mesh=_sc_mesh(),
        scratch_types=[
            pltpu.VMEM((CH, CHUNK), jnp.int32),
            pltpu.VMEM((NPAD,), jnp.float32),
        ],
        compiler_params=pltpu.CompilerParams(needs_layout_passes=False),
    )
    def k(col_hbm, out_hbm, colbuf, hist):
        cid = lax.axis_index("c")
        sid = lax.axis_index("s")
        wid = cid * NS + sid
        z16 = jnp.zeros((16,), jnp.float32)

        def zero_body(i, carry):
            hist[pl.ds(i * 16, 16)] = z16
            return carry

        lax.fori_loop(0, NPAD // 16, zero_body, 0)
        pltpu.sync_copy(col_hbm.at[wid], colbuf)
        ones16 = jnp.ones((16,), jnp.float32)

        def edge_body(e, carry):
            j = e // (CHUNK // 16)
            q = e % (CHUNK // 16)
            idx = colbuf[j, pl.ds(q * 16, 16)]
            plsc.addupdate_scatter(hist, [idx], ones16)
            return carry

        lax.fori_loop(0, CH * (CHUNK // 16), edge_body, 0)
        pltpu.sync_copy(hist, out_hbm.at[wid])

    return k(col3)


def _dense(xp, mp, W, parts):
    def body(x_ref, m_ref, w_ref, p_ref, h2_ref, dis_ref):
        ones = jnp.ones((NW, 1), jnp.float32)
        deg = (
            lax.dot_general(
                p_ref[...], ones, (((0,), (0,)), ((), ())),
                preferred_element_type=jnp.float32,
            )
            + 1.0
        )
        dis = lax.rsqrt(deg)
        h = jnp.dot(
            x_ref[...] * m_ref[...], w_ref[...],
            preferred_element_type=jnp.float32,
        )
        h2_ref[...] = dis * h
        dis_ref[...] = dis

    return pl.pallas_call(
        body,
        grid=(NPAD // BN,),
        in_specs=[
            pl.BlockSpec((BN, C), lambda i: (i, 0)),
            pl.BlockSpec((BN, C), lambda i: (i, 0)),
            pl.BlockSpec((C, C), lambda i: (0, 0)),
            pl.BlockSpec((NW, BN), lambda i: (0, i)),
        ],
        out_specs=[
            pl.BlockSpec((BN, C), lambda i: (i, 0)),
            pl.BlockSpec((BN, 1), lambda i: (i, 0)),
        ],
        out_shape=[
            jax.ShapeDtypeStruct((NPAD, C), jnp.float32),
            jax.ShapeDtypeStruct((NPAD, 1), jnp.float32),
        ],
    )(xp, mp, W, parts)


def _edge_accumulate(h2, row3, col3):
    @functools.partial(
        pl.kernel,
        out_type=jax.ShapeDtypeStruct((NC, NPAD, C), jnp.float32),
        mesh=_sc_mesh(),
        scratch_types=[
            pltpu.VMEM((CH, CHUNK), jnp.int32),
            pltpu.VMEM((CH, CHUNK), jnp.int32),
            pltpu.VMEM((CHUNK, C), jnp.float32),
            pltpu.VMEM_SHARED((NPAD, C), jnp.float32),
            pltpu.SemaphoreType.DMA,
        ],
        compiler_params=pltpu.CompilerParams(needs_layout_passes=False),
    )
    def k(h2_hbm, row_hbm, col_hbm, out_hbm, rowbuf, colbuf, gbuf, acc, sem):
        cid = lax.axis_index("c")
        sid = lax.axis_index("s")
        wid = cid * NS + sid
        z16 = jnp.zeros((16,), jnp.float32)

        def zbuf_body(i, carry):
            r = i // (C // 16)
            q = i % (C // 16)
            gbuf[r, pl.ds(q * 16, 16)] = z16
            return carry

        lax.fori_loop(0, CHUNK * (C // 16), zbuf_body, 0)

        def zacc_body(i, carry):
            pltpu.sync_copy(gbuf, acc.at[pl.ds(sid * STRIPE + i * CHUNK, CHUNK)])
            return carry

        lax.fori_loop(0, STRIPE // CHUNK, zacc_body, 0)
        pltpu.sync_copy(row_hbm.at[wid], rowbuf)
        pltpu.sync_copy(col_hbm.at[wid], colbuf)
        plsc.subcore_barrier()

        def edge_body(j, carry):
            pltpu.async_copy(h2_hbm.at[rowbuf.at[j]], gbuf, sem).wait()
            pltpu.sync_copy(gbuf, acc.at[colbuf.at[j]], add=True)
            return carry

        lax.fori_loop(0, CH, edge_body, 0)
        plsc.subcore_barrier()
        pltpu.sync_copy(
            acc.at[pl.ds(sid * STRIPE, STRIPE)],
            out_hbm.at[cid, pl.ds(sid * STRIPE, STRIPE)],
        )

    return k(h2, row3, col3)


def _finalize(acc_parts, h2, dis, b2):
    def body(a_ref, h2_ref, dis_ref, b_ref, y_ref):
        s = a_ref[0] + a_ref[1] + h2_ref[...]
        out = jnp.maximum(dis_ref[...] * s + b_ref[...], 0.0)
        nrm = jnp.sqrt(jnp.sum(out * out, axis=1, keepdims=True))
        y_ref[...] = out / jnp.maximum(nrm, 1e-12)

    return pl.pallas_call(
        body,
        grid=(NPAD // BN,),
        in_specs=[
            pl.BlockSpec((NC, BN, C), lambda i: (0, i, 0)),
            pl.BlockSpec((BN, C), lambda i: (i, 0)),
            pl.BlockSpec((BN, 1), lambda i: (i, 0)),
            pl.BlockSpec((1, C), lambda i: (0, 0)),
        ],
        out_specs=pl.BlockSpec((BN, C), lambda i: (i, 0)),
        out_shape=jax.ShapeDtypeStruct((NPAD, C), jnp.float32),
    )(acc_parts, h2, dis, b2)


def kernel(x, edge_index, W, b):
    xp = jnp.pad(x.astype(jnp.float32), ((0, NPAD - N_NODES), (0, 0)))
    mp = _mask_scale()
    pad_idx = N_NODES + jnp.arange(EPAD - N_EDGES, dtype=jnp.int32) % (
        NPAD - N_NODES
    )
    rowp = jnp.concatenate([edge_index[0], pad_idx]).reshape(NW, CH, CHUNK)
    colp = jnp.concatenate([edge_index[1], pad_idx]).reshape(NW, CH, CHUNK)

    parts = _deg_partials(colp)
    h2, dis = _dense(xp, mp, W.astype(jnp.float32), parts)
    acc_parts = _edge_accumulate(h2, rowp, colp)
    y = _finalize(acc_parts, h2, dis, b.reshape(1, C).astype(jnp.float32))
    return y[:N_NODES]


# speedup vs baseline: 2.9148x; 1.3463x over previous
import functools

import numpy as np
import jax
import jax.numpy as jnp
from jax import lax
from jax.experimental import pallas as pl
from jax.experimental.pallas import tpu as pltpu
from jax.experimental.pallas import tpu_sc as plsc

N_NODES = 10000
C = 128
N_EDGES = 320000
NPAD = 10240
NC, NS = 2, 16
NW = NC * NS
CHUNK = 128
CH = 80
SLAB = 16
GROUP = 2
NSLAB = CH // SLAB
EPT = CH * CHUNK
EPAD = NW * EPT
DUMMY = NPAD - 1
BN = 1280
STRIPE = NPAD // NS

def _mask_scale():
    keep = jax.random.bernoulli(jax.random.key(42), 1.0 - 0.2, (N_NODES, C))
    scale = jnp.float32(1.0) / jnp.float32(1.0 - 0.2)
    m = jnp.where(keep, scale, jnp.float32(0.0))
    return jnp.pad(m, ((0, NPAD - N_NODES), (0, 0)))


def _sc_mesh():
    return plsc.VectorSubcoreMesh(
        core_axis_name="c", subcore_axis_name="s", num_cores=NC, num_subcores=NS
    )


def _deg_partials(col3):
    @functools.partial(
        pl.kernel,
        out_type=jax.ShapeDtypeStruct((NW, NPAD), jnp.float32),
        mesh=_sc_mesh(),
        scratch_types=[
            pltpu.VMEM((CH, CHUNK), jnp.int32),
            pltpu.VMEM((NPAD,), jnp.float32),
        ],
        compiler_params=pltpu.CompilerParams(needs_layout_passes=False),
    )
    def k(col_hbm, out_hbm, colbuf, hist):
        cid = lax.axis_index("c")
        sid = lax.axis_index("s")
        wid = cid * NS + sid
        z16 = jnp.zeros((16,), jnp.float32)

        def zero_body(i, carry):
            hist[pl.ds(i * 16, 16)] = z16
            return carry

        lax.fori_loop(0, NPAD // 16, zero_body, 0)
        pltpu.sync_copy(col_hbm.at[wid], colbuf)
        ones16 = jnp.ones((16,), jnp.float32)

        def edge_body(e, carry):
            j = e // (CHUNK // 16)
            q = e % (CHUNK // 16)
            idx = colbuf[j, pl.ds(q * 16, 16)]
            plsc.addupdate_scatter(hist, [idx], ones16)
            return carry

        lax.fori_loop(0, CH * (CHUNK // 16), edge_body, 0)
        pltpu.sync_copy(hist, out_hbm.at[wid])

    return k(col3)


def _dense(xp, mp, W, parts):
    def body(x_ref, m_ref, w_ref, p_ref, h2_ref, dis_ref):
        ones = jnp.ones((NW, 1), jnp.float32)
        deg = (
            lax.dot_general(
                p_ref[...], ones, (((0,), (0,)), ((), ())),
                preferred_element_type=jnp.float32,
            )
            + 1.0
        )
        dis = lax.rsqrt(deg)
        h = jnp.dot(
            x_ref[...] * m_ref[...], w_ref[...],
            preferred_element_type=jnp.float32,
        )
        h2_ref[...] = dis * h
        dis_ref[...] = dis

    return pl.pallas_call(
        body,
        grid=(NPAD // BN,),
        in_specs=[
            pl.BlockSpec((BN, C), lambda i: (i, 0)),
            pl.BlockSpec((BN, C), lambda i: (i, 0)),
            pl.BlockSpec((C, C), lambda i: (0, 0)),
            pl.BlockSpec((NW, BN), lambda i: (0, i)),
        ],
        out_specs=[
            pl.BlockSpec((BN, C), lambda i: (i, 0)),
            pl.BlockSpec((BN, 1), lambda i: (i, 0)),
        ],
        out_shape=[
            jax.ShapeDtypeStruct((NPAD, C), jnp.float32),
            jax.ShapeDtypeStruct((NPAD, 1), jnp.float32),
        ],
    )(xp, mp, W, parts)


def _edge_accumulate(h2, row3, col3):
    @functools.partial(
        pl.kernel,
        out_type=jax.ShapeDtypeStruct((NC, NPAD, C), jnp.float32),
        mesh=_sc_mesh(),
        scratch_types=[
            pltpu.VMEM((2, SLAB, CHUNK), jnp.int32),
            pltpu.VMEM((2, SLAB, CHUNK), jnp.int32),
            pltpu.VMEM((2, CHUNK, C), jnp.float32),
            pltpu.VMEM_SHARED((NPAD, C), jnp.float32),
            pltpu.SemaphoreType.DMA,
        ],
        compiler_params=pltpu.CompilerParams(needs_layout_passes=False),
    )
    def k(h2_hbm, row_hbm, col_hbm, out_hbm, rowbuf, colbuf, gbuf2, acc, sem):
        cid = lax.axis_index("c")
        sid = lax.axis_index("s")
        wid = cid * NS + sid
        z16 = jnp.zeros((16,), jnp.float32)

        def zbuf_body(i, carry):
            r = i // (C // 16)
            q = i % (C // 16)
            gbuf2[0, r, pl.ds(q * 16, 16)] = z16
            return carry

        lax.fori_loop(0, CHUNK * (C // 16), zbuf_body, 0)

        def zacc_body(i, carry):
            pltpu.sync_copy(
                gbuf2.at[0], acc.at[pl.ds(sid * STRIPE + i * CHUNK, CHUNK)]
            )
            return carry

        lax.fori_loop(0, STRIPE // CHUNK, zacc_body, 0)

        def load_slab(s):
            buf = s % 2
            pltpu.sync_copy(row_hbm.at[wid, pl.ds(s * SLAB, SLAB)], rowbuf.at[buf])
            pltpu.sync_copy(col_hbm.at[wid, pl.ds(s * SLAB, SLAB)], colbuf.at[buf])

        load_slab(0)
        plsc.subcore_barrier()

        def start_gather(j):
            pltpu.async_copy(
                h2_hbm.at[rowbuf.at[(j // SLAB) % 2, j % SLAB]],
                gbuf2.at[j % 2],
                sem,
            )

        for j in range(2):
            start_gather(j)

        def edge_body(j, carry):
            @pl.when(jnp.logical_and(j % SLAB == 0, (j // SLAB) + 1 < NSLAB))
            def _():
                load_slab((j // SLAB) + 1)

            slot = j % 2
            pltpu.make_async_copy(
                h2_hbm.at[rowbuf.at[(j // SLAB) % 2, j % SLAB]],
                gbuf2.at[slot],
                sem,
            ).wait()
            pltpu.sync_copy(
                gbuf2.at[slot],
                acc.at[colbuf.at[(j // SLAB) % 2, j % SLAB]],
                add=True,
            )

            @pl.when(j + 2 < CH)
            def _():
                start_gather(j + 2)

            return carry

        lax.fori_loop(0, CH, edge_body, 0)
        plsc.subcore_barrier()
        pltpu.sync_copy(
            acc.at[pl.ds(sid * STRIPE, STRIPE)],
            out_hbm.at[cid, pl.ds(sid * STRIPE, STRIPE)],
        )

    return k(h2, row3, col3)


def _finalize(acc_parts, h2, dis, b2):
    def body(a_ref, h2_ref, dis_ref, b_ref, y_ref):
        s = a_ref[0] + a_ref[1] + h2_ref[...]
        out = jnp.maximum(dis_ref[...] * s + b_ref[...], 0.0)
        nrm = jnp.sqrt(jnp.sum(out * out, axis=1, keepdims=True))
        y_ref[...] = out / jnp.maximum(nrm, 1e-12)

    return pl.pallas_call(
        body,
        grid=(NPAD // BN,),
        in_specs=[
            pl.BlockSpec((NC, BN, C), lambda i: (0, i, 0)),
            pl.BlockSpec((BN, C), lambda i: (i, 0)),
            pl.BlockSpec((BN, 1), lambda i: (i, 0)),
            pl.BlockSpec((1, C), lambda i: (0, 0)),
        ],
        out_specs=pl.BlockSpec((BN, C), lambda i: (i, 0)),
        out_shape=jax.ShapeDtypeStruct((NPAD, C), jnp.float32),
    )(acc_parts, h2, dis, b2)


def kernel(x, edge_index, W, b):
    xp = jnp.pad(x.astype(jnp.float32), ((0, NPAD - N_NODES), (0, 0)))
    mp = _mask_scale()
    pad_idx = N_NODES + jnp.arange(EPAD - N_EDGES, dtype=jnp.int32) % (
        NPAD - N_NODES
    )
    rowp = jnp.concatenate([edge_index[0], pad_idx]).reshape(NW, CH, CHUNK)
    colp = jnp.concatenate([edge_index[1], pad_idx]).reshape(NW, CH, CHUNK)

    parts = _deg_partials(colp)
    h2, dis = _dense(xp, mp, W.astype(jnp.float32), parts)
    acc_parts = _edge_accumulate(h2, rowp, colp)
    y = _finalize(acc_parts, h2, dis, b.reshape(1, C).astype(jnp.float32))
    return y[:N_NODES]
